# baseline (speedup 1.0000x reference)
import functools

import jax
import jax.numpy as jnp
import numpy as np
from jax import lax
from jax.experimental import pallas as pl
from jax.experimental.pallas import tpu as pltpu
from jax.experimental.pallas import tpu_sc as plsc

_HASH_PRIMES = [2654435761, 2246822519, 3266489917, 2028178513,
                1220703125, 1610612741, 805306457, 402653189]
_SKIP_PATTERNS = ((1, 2), (1, 2, 3), (1, 3), (2, 3))
_BUCKETS = 16384
_NUM_TABLES = 4

def _wrap(p):
    return int(np.int32(np.uint32(p)))

_TBL_PRIMES = []
for _t in range(_NUM_TABLES):
    _d = {}
    for _k, _off in enumerate(_SKIP_PATTERNS[_t]):
        _d[_off] = _wrap(_HASH_PRIMES[(_t * 3 + _k) % len(_HASH_PRIMES)])
    _TBL_PRIMES.append(_d)

_B, _S = 64, 512
_N = _B * _S
_D_BYTE = 64
_D_HASH = 32
_HID = 512
_VOCAB = 1024

_CHUNK = 128


def _sc_hash_gather(s1, s2, s3, byte_idx, byte_table, t0, t1, t2, t3):
    info = plsc.get_sparse_core_info()
    nc, ns = info.num_cores, info.num_subcores
    nw = nc * ns
    tok_per_w = _N // nw
    n_chunks = tok_per_w // _CHUNK

    mesh = plsc.VectorSubcoreMesh(core_axis_name="c", subcore_axis_name="s")

    out_type = [
        jax.ShapeDtypeStruct((_N, _D_BYTE), jnp.float32),
        jax.ShapeDtypeStruct((_N, _D_HASH), jnp.float32),
        jax.ShapeDtypeStruct((_N, _D_HASH), jnp.float32),
        jax.ShapeDtypeStruct((_N, _D_HASH), jnp.float32),
        jax.ShapeDtypeStruct((_N, _D_HASH), jnp.float32),
    ]
    scratch_types = [
        pltpu.VMEM((_CHUNK,), jnp.int32),
        pltpu.VMEM((_CHUNK,), jnp.int32),
        pltpu.VMEM((_CHUNK,), jnp.int32),
        pltpu.VMEM((_CHUNK,), jnp.int32),
        pltpu.VMEM((_CHUNK,), jnp.int32),
        pltpu.VMEM((_CHUNK,), jnp.int32),
        pltpu.VMEM((_CHUNK,), jnp.int32),
        pltpu.VMEM((_CHUNK,), jnp.int32),
        pltpu.VMEM((_CHUNK, _D_BYTE), jnp.float32),
        pltpu.VMEM((_CHUNK, _D_HASH), jnp.float32),
        pltpu.VMEM((_CHUNK, _D_HASH), jnp.float32),
        pltpu.VMEM((_CHUNK, _D_HASH), jnp.float32),
        pltpu.VMEM((_CHUNK, _D_HASH), jnp.float32),
        pltpu.SemaphoreType.DMA,
    ]

    @functools.partial(
        pl.kernel, mesh=mesh, out_type=out_type, scratch_types=scratch_types,
        compiler_params=pltpu.CompilerParams(use_tc_tiling_on_sc=False))
    def k(s1_h, s2_h, s3_h, bi_h, bt_h, t0_h, t1_h, t2_h, t3_h,
          byte_o, h0_o, h1_o, h2_o, h3_o,
          s1_v, s2_v, s3_v, bi_v, i0_v, i1_v, i2_v, i3_v,
          br_v, r0_v, r1_v, r2_v, r3_v, sem):
        wid = lax.axis_index("s") * jnp.int32(nc) + lax.axis_index("c")
        base = wid * jnp.int32(tok_per_w)

        def chunk_body(c):
            off = base + jnp.int32(c * _CHUNK)
            sl = pl.ds(off, _CHUNK)
            pltpu.sync_copy(s1_h.at[sl], s1_v)
            pltpu.sync_copy(s2_h.at[sl], s2_v)
            pltpu.sync_copy(s3_h.at[sl], s3_v)
            pltpu.sync_copy(bi_h.at[sl], bi_v)
            for i in range(_CHUNK // 16):
                vs = pl.ds(i * 16, 16)
                a1 = s1_v[vs]
                a2 = s2_v[vs]
                a3 = s3_v[vs]
                p = _TBL_PRIMES[0]
                i0_v[vs] = ((a1 * p[1]) ^ (a2 * p[2])) & (_BUCKETS - 1)
                p = _TBL_PRIMES[1]
                i1_v[vs] = ((a1 * p[1]) ^ (a2 * p[2]) ^ (a3 * p[3])) & (_BUCKETS - 1)
                p = _TBL_PRIMES[2]
                i2_v[vs] = ((a1 * p[1]) ^ (a3 * p[3])) & (_BUCKETS - 1)
                p = _TBL_PRIMES[3]
                i3_v[vs] = ((a2 * p[2]) ^ (a3 * p[3])) & (_BUCKETS - 1)
            cps = [
                pltpu.async_copy(bt_h.at[bi_v], br_v, sem),
                pltpu.async_copy(t0_h.at[i0_v], r0_v, sem),
                pltpu.async_copy(t1_h.at[i1_v], r1_v, sem),
                pltpu.async_copy(t2_h.at[i2_v], r2_v, sem),
                pltpu.async_copy(t3_h.at[i3_v], r3_v, sem),
            ]
            for cp in cps:
                cp.wait()
            pltpu.sync_copy(br_v, byte_o.at[sl])
            pltpu.sync_copy(r0_v, h0_o.at[sl])
            pltpu.sync_copy(r1_v, h1_o.at[sl])
            pltpu.sync_copy(r2_v, h2_o.at[sl])
            pltpu.sync_copy(r3_v, h3_o.at[sl])

        for c in range(n_chunks):
            chunk_body(c)

    return k(s1, s2, s3, byte_idx, byte_table, t0, t1, t2, t3)


_TOK_BLK = 512


def _tc_mlp_body(bp, h0, h1, h2, h3, w1b, w10, w11, w12, w13, b1r, w2, b2r,
                 out):
    bf = jnp.bfloat16

    def dot(a, b):
        return jnp.dot(a.astype(bf), b.astype(bf),
                       preferred_element_type=jnp.float32)

    acc = dot(bp[...], w1b[...])
    acc += dot(h0[...], w10[...])
    acc += dot(h1[...], w11[...])
    acc += dot(h2[...], w12[...])
    acc += dot(h3[...], w13[...])
    h = jnp.maximum(acc + b1r[...], 0.0)
    out[...] = dot(h, w2[...]) + b2r[...]


def _tc_mlp(bp, h0, h1, h2, h3, w1b, w10, w11, w12, w13, b1, w2, b2):
    n_blk = _N // _TOK_BLK
    grid = (n_blk,)
    def _blk(i):
        return (i, np.int32(0))

    def _full(i):
        return (np.int32(0), np.int32(0))

    in_specs = [
        pl.BlockSpec((_TOK_BLK, _D_BYTE), _blk),
        pl.BlockSpec((_TOK_BLK, _D_HASH), _blk),
        pl.BlockSpec((_TOK_BLK, _D_HASH), _blk),
        pl.BlockSpec((_TOK_BLK, _D_HASH), _blk),
        pl.BlockSpec((_TOK_BLK, _D_HASH), _blk),
        pl.BlockSpec((_D_BYTE, _HID), _full),
        pl.BlockSpec((_D_HASH, _HID), _full),
        pl.BlockSpec((_D_HASH, _HID), _full),
        pl.BlockSpec((_D_HASH, _HID), _full),
        pl.BlockSpec((_D_HASH, _HID), _full),
        pl.BlockSpec((1, _HID), _full),
        pl.BlockSpec((_HID, _VOCAB), _full),
        pl.BlockSpec((1, _VOCAB), _full),
    ]
    out_spec = pl.BlockSpec((_TOK_BLK, _VOCAB), _blk)
    return pl.pallas_call(
        _tc_mlp_body,
        grid=grid,
        in_specs=in_specs,
        out_specs=out_spec,
        out_shape=jax.ShapeDtypeStruct((_N, _VOCAB), jnp.float32),
        compiler_params=pltpu.CompilerParams(
            dimension_semantics=("arbitrary",)),
    )(bp, h0, h1, h2, h3, w1b, w10, w11, w12, w13, b1, w2, b2)


def kernel(chars, byte_table, hash_tables, W1, b1, W2, b2):
    chars32 = chars.astype(jnp.int32)
    s1 = jnp.pad(chars32[:, :-1], ((0, 0), (1, 0))).reshape(_N)
    s2 = jnp.pad(chars32[:, :-2], ((0, 0), (2, 0))).reshape(_N)
    s3 = jnp.pad(chars32[:, :-3], ((0, 0), (3, 0))).reshape(_N)
    bi = chars32.reshape(_N)

    bt = byte_table.astype(jnp.float32)
    t0, t1, t2, t3 = (hash_tables[i] for i in range(4))

    bp, h0, h1, h2, h3 = _sc_hash_gather(s1, s2, s3, bi, bt, t0, t1, t2, t3)

    w1b = W1[:_D_BYTE]
    w10 = W1[_D_BYTE:_D_BYTE + _D_HASH]
    w11 = W1[_D_BYTE + _D_HASH:_D_BYTE + 2 * _D_HASH]
    w12 = W1[_D_BYTE + 2 * _D_HASH:_D_BYTE + 3 * _D_HASH]
    w13 = W1[_D_BYTE + 3 * _D_HASH:]
    out = _tc_mlp(bp, h0, h1, h2, h3, w1b, w10, w11, w12, w13,
                  b1.reshape(1, _HID), W2, b2.reshape(1, _VOCAB))
    return out.reshape(_B, _S, _VOCAB)


# speedup vs baseline: 1.0621x; 1.0621x over previous
import functools

import jax
import jax.numpy as jnp
import numpy as np
from jax import lax
from jax.experimental import pallas as pl
from jax.experimental.pallas import tpu as pltpu
from jax.experimental.pallas import tpu_sc as plsc

_HASH_PRIMES = [2654435761, 2246822519, 3266489917, 2028178513,
                1220703125, 1610612741, 805306457, 402653189]
_SKIP_PATTERNS = ((1, 2), (1, 2, 3), (1, 3), (2, 3))
_BUCKETS = 16384
_NUM_TABLES = 4

def _wrap(p):
    return int(np.int32(np.uint32(p)))

_TBL_PRIMES = []
for _t in range(_NUM_TABLES):
    _d = {}
    for _k, _off in enumerate(_SKIP_PATTERNS[_t]):
        _d[_off] = _wrap(_HASH_PRIMES[(_t * 3 + _k) % len(_HASH_PRIMES)])
    _TBL_PRIMES.append(_d)

_B, _S = 64, 512
_N = _B * _S
_D_BYTE = 64
_D_HASH = 32
_HID = 512
_VOCAB = 1024

_CHUNK = 128


def _sc_hash_gather(s1, s2, s3, byte_idx, byte_table, t0, t1, t2, t3):
    info = plsc.get_sparse_core_info()
    nc, ns = info.num_cores, info.num_subcores
    nw = nc * ns
    tok_per_w = _N // nw
    n_chunks = tok_per_w // _CHUNK

    mesh = plsc.VectorSubcoreMesh(core_axis_name="c", subcore_axis_name="s")

    out_type = [
        jax.ShapeDtypeStruct((_N, _D_BYTE), jnp.float32),
        jax.ShapeDtypeStruct((_N, _D_HASH), jnp.float32),
        jax.ShapeDtypeStruct((_N, _D_HASH), jnp.float32),
        jax.ShapeDtypeStruct((_N, _D_HASH), jnp.float32),
        jax.ShapeDtypeStruct((_N, _D_HASH), jnp.float32),
    ]
    qtok = 2 * _CHUNK
    n_q = tok_per_w // qtok

    scratch_types = [
        pltpu.VMEM((tok_per_w,), jnp.int32),
        pltpu.VMEM((tok_per_w,), jnp.int32),
        pltpu.VMEM((tok_per_w,), jnp.int32),
        pltpu.VMEM((tok_per_w,), jnp.int32),
        pltpu.VMEM((tok_per_w,), jnp.int32),
        pltpu.VMEM((tok_per_w,), jnp.int32),
        pltpu.VMEM((tok_per_w,), jnp.int32),
        pltpu.VMEM((tok_per_w,), jnp.int32),
        pltpu.VMEM((2, qtok, _D_BYTE), jnp.float32),
        pltpu.VMEM((2, qtok, _D_HASH), jnp.float32),
        pltpu.VMEM((2, qtok, _D_HASH), jnp.float32),
        pltpu.VMEM((2, qtok, _D_HASH), jnp.float32),
        pltpu.VMEM((2, qtok, _D_HASH), jnp.float32),
        pltpu.SemaphoreType.DMA,
        pltpu.SemaphoreType.DMA,
        pltpu.SemaphoreType.DMA,
        pltpu.SemaphoreType.DMA,
    ]

    @functools.partial(
        pl.kernel, mesh=mesh, out_type=out_type, scratch_types=scratch_types,
        compiler_params=pltpu.CompilerParams(use_tc_tiling_on_sc=False))
    def k(s1_h, s2_h, s3_h, bi_h, bt_h, t0_h, t1_h, t2_h, t3_h,
          byte_o, h0_o, h1_o, h2_o, h3_o,
          s1_v, s2_v, s3_v, bi_v, i0_v, i1_v, i2_v, i3_v,
          br_v, r0_v, r1_v, r2_v, r3_v, sg0, sg1, sw0, sw1):
        wid = lax.axis_index("s") * jnp.int32(nc) + lax.axis_index("c")
        base = wid * jnp.int32(tok_per_w)
        sg = [sg0, sg1]
        sw = [sw0, sw1]

        pltpu.sync_copy(s1_h.at[pl.ds(base, tok_per_w)], s1_v)
        pltpu.sync_copy(s2_h.at[pl.ds(base, tok_per_w)], s2_v)
        pltpu.sync_copy(s3_h.at[pl.ds(base, tok_per_w)], s3_v)
        pltpu.sync_copy(bi_h.at[pl.ds(base, tok_per_w)], bi_v)
        for i in range(tok_per_w // 16):
            vs = pl.ds(i * 16, 16)
            a1 = s1_v[vs]
            a2 = s2_v[vs]
            a3 = s3_v[vs]
            p = _TBL_PRIMES[0]
            i0_v[vs] = ((a1 * p[1]) ^ (a2 * p[2])) & (_BUCKETS - 1)
            p = _TBL_PRIMES[1]
            i1_v[vs] = ((a1 * p[1]) ^ (a2 * p[2]) ^ (a3 * p[3])) & (_BUCKETS - 1)
            p = _TBL_PRIMES[2]
            i2_v[vs] = ((a1 * p[1]) ^ (a3 * p[3])) & (_BUCKETS - 1)
            p = _TBL_PRIMES[3]
            i3_v[vs] = ((a2 * p[2]) ^ (a3 * p[3])) & (_BUCKETS - 1)

        def fire_gathers(q):
            b = q % 2
            cps = []
            for kk in range(qtok // _CHUNK):
                isl = pl.ds((q * qtok // _CHUNK + kk) * _CHUNK, _CHUNK)
                dsl = pl.ds(kk * _CHUNK, _CHUNK)
                cps += [
                    pltpu.async_copy(bt_h.at[bi_v.at[isl]], br_v.at[np.int32(b), dsl], sg[b]),
                    pltpu.async_copy(t0_h.at[i0_v.at[isl]], r0_v.at[np.int32(b), dsl], sg[b]),
                    pltpu.async_copy(t1_h.at[i1_v.at[isl]], r1_v.at[np.int32(b), dsl], sg[b]),
                    pltpu.async_copy(t2_h.at[i2_v.at[isl]], r2_v.at[np.int32(b), dsl], sg[b]),
                    pltpu.async_copy(t3_h.at[i3_v.at[isl]], r3_v.at[np.int32(b), dsl], sg[b]),
                ]
            return cps

        def fire_writes(q):
            b = q % 2
            osl = pl.ds(base + jnp.int32(q * qtok), qtok)
            return [
                pltpu.async_copy(br_v.at[np.int32(b)], byte_o.at[osl], sw[b]),
                pltpu.async_copy(r0_v.at[np.int32(b)], h0_o.at[osl], sw[b]),
                pltpu.async_copy(r1_v.at[np.int32(b)], h1_o.at[osl], sw[b]),
                pltpu.async_copy(r2_v.at[np.int32(b)], h2_o.at[osl], sw[b]),
                pltpu.async_copy(r3_v.at[np.int32(b)], h3_o.at[osl], sw[b]),
            ]

        pend_g = {}
        pend_w = {}
        for q in range(n_q):
            if q >= 2:
                for cp in pend_w.pop(q - 2):
                    cp.wait()
            pend_g[q] = fire_gathers(q)
            if q >= 1:
                for cp in pend_g.pop(q - 1):
                    cp.wait()
                pend_w[q - 1] = fire_writes(q - 1)
        for cp in pend_g.pop(n_q - 1):
            cp.wait()
        pend_w[n_q - 1] = fire_writes(n_q - 1)
        for q in (n_q - 2, n_q - 1):
            for cp in pend_w.pop(q):
                cp.wait()

    return k(s1, s2, s3, byte_idx, byte_table, t0, t1, t2, t3)


_TOK_BLK = 512


def _tc_mlp_body(bp, h0, h1, h2, h3, w1b, w10, w11, w12, w13, b1r, w2, b2r,
                 out):
    bf = jnp.bfloat16

    def dot(a, b):
        return jnp.dot(a.astype(bf), b.astype(bf),
                       preferred_element_type=jnp.float32)

    acc = dot(bp[...], w1b[...])
    acc += dot(h0[...], w10[...])
    acc += dot(h1[...], w11[...])
    acc += dot(h2[...], w12[...])
    acc += dot(h3[...], w13[...])
    h = jnp.maximum(acc + b1r[...], 0.0)
    out[...] = dot(h, w2[...]) + b2r[...]


def _tc_mlp(bp, h0, h1, h2, h3, w1b, w10, w11, w12, w13, b1, w2, b2):
    n_blk = _N // _TOK_BLK
    grid = (n_blk,)
    def _blk(i):
        return (i, np.int32(0))

    def _full(i):
        return (np.int32(0), np.int32(0))

    in_specs = [
        pl.BlockSpec((_TOK_BLK, _D_BYTE), _blk),
        pl.BlockSpec((_TOK_BLK, _D_HASH), _blk),
        pl.BlockSpec((_TOK_BLK, _D_HASH), _blk),
        pl.BlockSpec((_TOK_BLK, _D_HASH), _blk),
        pl.BlockSpec((_TOK_BLK, _D_HASH), _blk),
        pl.BlockSpec((_D_BYTE, _HID), _full),
        pl.BlockSpec((_D_HASH, _HID), _full),
        pl.BlockSpec((_D_HASH, _HID), _full),
        pl.BlockSpec((_D_HASH, _HID), _full),
        pl.BlockSpec((_D_HASH, _HID), _full),
        pl.BlockSpec((1, _HID), _full),
        pl.BlockSpec((_HID, _VOCAB), _full),
        pl.BlockSpec((1, _VOCAB), _full),
    ]
    out_spec = pl.BlockSpec((_TOK_BLK, _VOCAB), _blk)
    return pl.pallas_call(
        _tc_mlp_body,
        grid=grid,
        in_specs=in_specs,
        out_specs=out_spec,
        out_shape=jax.ShapeDtypeStruct((_N, _VOCAB), jnp.float32),
        compiler_params=pltpu.CompilerParams(
            dimension_semantics=("arbitrary",)),
    )(bp, h0, h1, h2, h3, w1b, w10, w11, w12, w13, b1, w2, b2)


def kernel(chars, byte_table, hash_tables, W1, b1, W2, b2):
    chars32 = chars.astype(jnp.int32)
    s1 = jnp.pad(chars32[:, :-1], ((0, 0), (1, 0))).reshape(_N)
    s2 = jnp.pad(chars32[:, :-2], ((0, 0), (2, 0))).reshape(_N)
    s3 = jnp.pad(chars32[:, :-3], ((0, 0), (3, 0))).reshape(_N)
    bi = chars32.reshape(_N)

    bt = byte_table.astype(jnp.float32)
    t0, t1, t2, t3 = (hash_tables[i] for i in range(4))

    bp, h0, h1, h2, h3 = _sc_hash_gather(s1, s2, s3, bi, bt, t0, t1, t2, t3)

    w1b = W1[:_D_BYTE]
    w10 = W1[_D_BYTE:_D_BYTE + _D_HASH]
    w11 = W1[_D_BYTE + _D_HASH:_D_BYTE + 2 * _D_HASH]
    w12 = W1[_D_BYTE + 2 * _D_HASH:_D_BYTE + 3 * _D_HASH]
    w13 = W1[_D_BYTE + 3 * _D_HASH:]
    out = _tc_mlp(bp, h0, h1, h2, h3, w1b, w10, w11, w12, w13,
                  b1.reshape(1, _HID), W2, b2.reshape(1, _VOCAB))
    return out.reshape(_B, _S, _VOCAB)
